# baseline (speedup 1.0000x reference)
import functools

import jax
import jax.numpy as jnp
import numpy as np
from jax import lax
from jax.experimental import pallas as pl
from jax.experimental.pallas import tpu as pltpu
from jax.experimental.pallas import tpu_sc as plsc

N = 10000
E = 160000
D_IN = 256
D_OUT = 256
H = 128
HW_ = 64
NT = 16
NC = 2
CH = 64
NCHUNK = 160
G = 8
NG = NCHUNK // G
GE = G * CH
EPT = CH * NCHUNK
EPAD = NT * EPT
ROWS_PT = 624
OR = 16
NPASS = ROWS_PT // OR
TAIL0 = NT * ROWS_PT
TAIL = N - TAIL0
MM_BLK = 1000
MM_NB = N // MM_BLK

_COLS_A = np.array([128 * h + 32 * (w // 16) + (w % 16)
                    for h in range(NC) for w in range(HW_)])
_COLS_B = _COLS_A + 16


def _mm_body(x_ref, wa_ref, wb_ref, o_ref):
    ya = jnp.dot(x_ref[...], wa_ref[0], preferred_element_type=jnp.float32)
    yb = jnp.dot(x_ref[...], wb_ref[0], preferred_element_type=jnp.float32)
    ya = ya.astype(jnp.bfloat16).astype(jnp.float32)
    yb = yb.astype(jnp.bfloat16).astype(jnp.float32)
    ua = lax.shift_right_logical(lax.bitcast_convert_type(ya, jnp.uint32),
                                 jnp.uint32(16))
    ub = lax.bitcast_convert_type(yb, jnp.uint32) & jnp.uint32(0xFFFF0000)
    o_ref[...] = lax.bitcast_convert_type(ua | ub, jnp.int32)


def _matmul_halves(x, Wa, Wb):
    return pl.pallas_call(
        _mm_body,
        grid=(MM_NB, NC),
        in_specs=[
            pl.BlockSpec((MM_BLK, D_IN), lambda i, h: (i, 0)),
            pl.BlockSpec((1, D_IN, HW_), lambda i, h: (h, 0, 0)),
            pl.BlockSpec((1, D_IN, HW_), lambda i, h: (h, 0, 0)),
        ],
        out_specs=pl.BlockSpec((MM_BLK, HW_), lambda i, h: (h * MM_NB + i, 0)),
        out_shape=jax.ShapeDtypeStruct((NC * N, HW_), jnp.int32),
    )(x, Wa, Wb)


_MESH = plsc.VectorSubcoreMesh(core_axis_name="c", subcore_axis_name="s")


@functools.partial(
    pl.kernel,
    out_type=jax.ShapeDtypeStruct((N, D_OUT), jnp.float32),
    mesh=_MESH,
    compiler_params=pltpu.CompilerParams(use_tc_tiling_on_sc=False,
                                         needs_layout_passes=False),
    scratch_types=[
        pltpu.VMEM((4, CH, HW_), jnp.int32),
        pltpu.VMEM((2, CH, H), jnp.float32),
        pltpu.VMEM((2, G, CH), jnp.int32),
        pltpu.VMEM((2, G, CH), jnp.int32),
        pltpu.VMEM((2 * GE,), jnp.float32),
        pltpu.VMEM((OR, H), jnp.float32),
        pltpu.VMEM((NC, H), jnp.float32),
        pltpu.VMEM_SHARED((N, H), jnp.float32),
        pltpu.SemaphoreType.DMA,
        pltpu.SemaphoreType.DMA,
        pltpu.SemaphoreType.DMA,
        pltpu.SemaphoreType.DMA,
        pltpu.SemaphoreType.DMA,
        pltpu.SemaphoreType.DMA,
    ],
)
def _sc_aggregate(xw_hbm, srcb_hbm, dst_hbm, w_hbm, b_hbm, out_hbm,
                  gbuf, sbuf, srcg, dstg, wgf, obuf, bbuf, acc,
                  gsem0, gsem1, gsem2, gsem3, ssem0, ssem1):
    c = lax.axis_index("c")
    s = lax.axis_index("s")
    gsems = (gsem0, gsem1, gsem2, gsem3)
    ssems = (ssem0, ssem1)

    pltpu.sync_copy(b_hbm, bbuf)
    bvecs = [bbuf[c, pl.ds(16 * k, 16)] for k in range(8)]

    @pl.loop(0, OR)
    def _fill(r):
        for k in range(8):
            obuf[r, pl.ds(16 * k, 16)] = bvecs[k]

    row0 = s * ROWS_PT

    @pl.loop(0, NPASS)
    def _init(p):
        pltpu.sync_copy(obuf, acc.at[pl.ds(row0 + p * OR, OR)])

    @pl.when(s == NT - 1)
    def _init_tail():
        pltpu.sync_copy(obuf.at[pl.ds(0, TAIL)], acc.at[pl.ds(TAIL0, TAIL)])

    pltpu.sync_copy(srcb_hbm.at[c, s, pl.ds(0, G)], srcg.at[0])
    pltpu.sync_copy(dst_hbm.at[s, pl.ds(0, G)], dstg.at[0])
    pltpu.sync_copy(w_hbm.at[s, pl.ds(0, GE)], wgf.at[pl.ds(0, GE)])

    plsc.subcore_barrier()

    for k in range(3):
        pltpu.async_copy(xw_hbm.at[srcg.at[0, k]], gbuf.at[k], gsems[k])

    def _chunk(j, p):
        ghj = (j // G) % 2
        slj = j % G
        pn = (p + 3) % 4
        p2 = p % 2

        jn = j + 3

        @pl.when(jn < NCHUNK)
        def _prefetch():
            pltpu.async_copy(xw_hbm.at[srcg.at[(jn // G) % 2, jn % G]],
                             gbuf.at[pn], gsems[pn])

        pltpu.make_async_copy(xw_hbm.at[srcg.at[0, 0]], gbuf.at[p],
                              gsems[p]).wait()

        @pl.when(j >= 2)
        def _wait_scatter():
            pltpu.make_async_copy(sbuf.at[p2], acc.at[dstg.at[0, 0]],
                                  ssems[p2]).wait()

        woff = GE * ghj + CH * slj

        @pl.loop(0, CH // 16)
        def _scale(g16):
            wv = wgf[pl.ds(woff + 16 * g16, 16)]
            for r_ in range(16):
                ws = wv[r_]
                row = 16 * g16 + r_
                for v in range(4):
                    wi = gbuf[p, row, pl.ds(16 * v, 16)]
                    a = plsc.bitcast(lax.shift_left(wi, jnp.int32(16)),
                                     jnp.float32)
                    b_ = plsc.bitcast(wi & jnp.int32(-65536), jnp.float32)
                    sbuf[p2, row, pl.ds(32 * v, 16)] = a * ws
                    sbuf[p2, row, pl.ds(32 * v + 16, 16)] = b_ * ws

        pltpu.async_copy(sbuf.at[p2], acc.at[dstg.at[ghj, slj]],
                         ssems[p2], add=True)

    @pl.loop(0, NCHUNK, step=4)
    def _chunks(j0):
        @pl.when(jnp.logical_and(j0 % G == 4, j0 // G + 1 < NG))
        def _stage():
            gn = j0 // G + 1
            oth = gn % 2
            pltpu.sync_copy(srcb_hbm.at[c, s, pl.ds(G * gn, G)],
                            srcg.at[oth])
            pltpu.sync_copy(dst_hbm.at[s, pl.ds(G * gn, G)], dstg.at[oth])
            pltpu.sync_copy(w_hbm.at[s, pl.ds(GE * gn, GE)],
                            wgf.at[pl.ds(GE * oth, GE)])

        for k in range(4):
            _chunk(j0 + k, k)

    for k in range(2):
        pltpu.make_async_copy(sbuf.at[k], acc.at[dstg.at[0, 0]],
                              ssems[k]).wait()

    plsc.subcore_barrier()

    col0 = pl.multiple_of(c * H, H)
    pltpu.sync_copy(acc.at[pl.ds(row0, ROWS_PT)],
                    out_hbm.at[pl.ds(row0, ROWS_PT), pl.ds(col0, H)])

    @pl.when(s == NT - 1)
    def _copy_tail():
        pltpu.sync_copy(acc.at[pl.ds(TAIL0, TAIL)],
                        out_hbm.at[pl.ds(TAIL0, TAIL), pl.ds(col0, H)])


def kernel(x, edge_index, edge_weight, W, b):
    src = edge_index[1].astype(jnp.int32)
    dst = edge_index[0].astype(jnp.int32)
    w = edge_weight.astype(jnp.float32)

    pad = EPAD - E
    pad_idx = (jnp.arange(pad, dtype=jnp.int32) * 61) % N
    src_p = jnp.concatenate([src, pad_idx]).reshape(NT, NCHUNK, CH)
    dst_p = jnp.concatenate([dst, pad_idx]).reshape(NT, NCHUNK, CH)
    w_p = jnp.concatenate([w, jnp.zeros((pad,), jnp.float32)])
    w_p = w_p.reshape(NT, EPT)
    srcb = jnp.stack([src_p, src_p + N])
    b2 = b.reshape(NC, H)

    Wa = jnp.transpose(W[:, _COLS_A.reshape(NC, HW_)], (1, 0, 2))
    Wb = jnp.transpose(W[:, _COLS_B.reshape(NC, HW_)], (1, 0, 2))
    xw = _matmul_halves(x, Wa, Wb)
    return _sc_aggregate(xw, srcb, dst_p, w_p, b2)
